# baseline (speedup 1.0000x reference)
import functools

import jax
import jax.numpy as jnp
from jax import lax
from jax.experimental import pallas as pl
from jax.experimental.pallas import tpu as pltpu
from jax.experimental.pallas import tpu_sc as plsc

NC = 2
NS = 16
NW = NC * NS
BCH = 8
FBB = 512


LQ = 128


def _tc_pad_idx(indices, b, l):
    def body(in_ref, out_ref):
        x = in_ref[...].astype(jnp.int32)
        out_ref[...] = jnp.pad(x, ((0, 0), (0, LQ - l))).reshape(FBB * LQ)

    return pl.pallas_call(
        body,
        grid=(b // FBB,),
        in_specs=[pl.BlockSpec((FBB, l), lambda i: (i, 0))],
        out_specs=pl.BlockSpec((FBB * LQ,), lambda i: (i,)),
        out_shape=jax.ShapeDtypeStruct((b * LQ,), jnp.int32),
    )(indices)


def _sc_gather(idx_flat, table, b, l, d):
    lpad = -(-l // 8) * 8
    dpad = -(-d // 128) * 128
    per_w_b = b // NW
    chunk = BCH * LQ
    nchunk = per_w_b // BCH
    npair = nchunk // 2
    assert nchunk % 2 == 0 and nchunk >= 4
    mesh = plsc.VectorSubcoreMesh(core_axis_name="c", subcore_axis_name="s")

    @functools.partial(
        pl.kernel,
        out_type=jax.ShapeDtypeStruct((b, lpad, dpad), jnp.float32),
        mesh=mesh,
        scratch_types=[
            pltpu.VMEM((chunk,), jnp.int32),
            pltpu.VMEM((chunk,), jnp.int32),
            pltpu.VMEM((chunk, d), jnp.float32),
            pltpu.VMEM((chunk, d), jnp.float32),
            pltpu.SemaphoreType.DMA,
            pltpu.SemaphoreType.DMA,
            pltpu.SemaphoreType.DMA,
        ],
        compiler_params=pltpu.CompilerParams(use_tc_tiling_on_sc=False),
    )
    def k(table_hbm, idx_hbm, out_hbm, idx0, idx1, rows0, rows1, g0, g1, osem):
        wid = lax.axis_index("s") * NC + lax.axis_index("c")
        base = wid * per_w_b * LQ
        bbase = wid * per_w_b

        def idx_in(c, dst):
            pltpu.sync_copy(idx_hbm.at[pl.ds(base + c * chunk, chunk)], dst)

        def out_wr(c, src):
            b0 = bbase + c * BCH
            cps = [
                pltpu.make_async_copy(
                    src.at[pl.ds(j * LQ, l)],
                    out_hbm.at[b0 + j, pl.ds(0, l), pl.ds(0, d)],
                    osem,
                )
                for j in range(BCH)
            ]
            for cp in cps:
                cp.start()
            for cp in cps:
                cp.wait()

        idx_in(0, idx0)
        pltpu.async_copy(table_hbm.at[idx0], rows0, g0)

        @pl.loop(0, npair - 1)
        def _body(p):
            c = 2 * p
            idx_in(c + 1, idx1)
            pltpu.make_async_copy(table_hbm.at[idx0], rows0, g0).wait()
            pltpu.async_copy(table_hbm.at[idx1], rows1, g1)
            out_wr(c, rows0)
            idx_in(c + 2, idx0)
            pltpu.make_async_copy(table_hbm.at[idx1], rows1, g1).wait()
            pltpu.async_copy(table_hbm.at[idx0], rows0, g0)
            out_wr(c + 1, rows1)

        c = nchunk - 2
        idx_in(c + 1, idx1)
        pltpu.make_async_copy(table_hbm.at[idx0], rows0, g0).wait()
        pltpu.async_copy(table_hbm.at[idx1], rows1, g1)
        out_wr(c, rows0)
        pltpu.make_async_copy(table_hbm.at[idx1], rows1, g1).wait()
        out_wr(c + 1, rows1)

    return k(table, idx_flat)


@functools.partial(jax.jit, static_argnums=(2, 3, 4))
def _embed(indices, table, b, l, d):
    idx_flat = _tc_pad_idx(indices, b, l)
    out = _sc_gather(idx_flat, table, b, l, d)
    return out[:, :l, :d]


def kernel(indices, table):
    b, l = indices.shape
    d = table.shape[1]
    return _embed(indices, table, b, l, d)


# speedup vs baseline: 5.9031x; 5.9031x over previous
import functools

import jax
import jax.numpy as jnp
from jax import lax
from jax.experimental import pallas as pl
from jax.experimental.pallas import tpu as pltpu
from jax.experimental.pallas import tpu_sc as plsc

NC = 2
NS = 16
NW = NC * NS
BCH = 8
FBB = 512


LQ = 128


def _tc_pad_idx(indices, b, l, vocab):
    def body(in_ref, out_ref):
        x = in_ref[...].astype(jnp.int32)
        i = pl.program_id(0)
        row = jax.lax.broadcasted_iota(jnp.int32, (FBB, LQ), 0) + i * FBB
        lane = jax.lax.broadcasted_iota(jnp.int32, (FBB, LQ), 1)
        filler = (row * LQ + lane) % vocab
        padded = jnp.pad(x, ((0, 0), (0, LQ - l)))
        out_ref[...] = jnp.where(lane < l, padded, filler).reshape(FBB * LQ)

    return pl.pallas_call(
        body,
        grid=(b // FBB,),
        in_specs=[pl.BlockSpec((FBB, l), lambda i: (i, 0))],
        out_specs=pl.BlockSpec((FBB * LQ,), lambda i: (i,)),
        out_shape=jax.ShapeDtypeStruct((b * LQ,), jnp.int32),
    )(indices)


def _sc_gather(idx_flat, table, b, l, d):
    lpad = -(-l // 8) * 8
    dpad = -(-d // 128) * 128
    per_w_b = b // NW
    chunk = BCH * LQ
    nchunk = per_w_b // BCH
    npair = nchunk // 2
    assert nchunk % 2 == 0 and nchunk >= 4
    mesh = plsc.VectorSubcoreMesh(core_axis_name="c", subcore_axis_name="s")

    @functools.partial(
        pl.kernel,
        out_type=jax.ShapeDtypeStruct((b, lpad, dpad), jnp.float32),
        mesh=mesh,
        scratch_types=[
            pltpu.VMEM((chunk,), jnp.int32),
            pltpu.VMEM((chunk,), jnp.int32),
            pltpu.VMEM((chunk, d), jnp.float32),
            pltpu.VMEM((chunk, d), jnp.float32),
            pltpu.SemaphoreType.DMA,
            pltpu.SemaphoreType.DMA,
            pltpu.SemaphoreType.DMA,
        ],
        compiler_params=pltpu.CompilerParams(use_tc_tiling_on_sc=False),
    )
    def k(table_hbm, idx_hbm, out_hbm, idx0, idx1, rows0, rows1, g0, g1, osem):
        wid = lax.axis_index("s") * NC + lax.axis_index("c")
        base = wid * per_w_b * LQ
        bbase = wid * per_w_b

        def idx_in(c, dst):
            pltpu.sync_copy(idx_hbm.at[pl.ds(base + c * chunk, chunk)], dst)

        def out_wr(c, src):
            b0 = bbase + c * BCH
            cps = [
                pltpu.make_async_copy(
                    src.at[pl.ds(j * LQ, l)],
                    out_hbm.at[b0 + j, pl.ds(0, l), pl.ds(0, d)],
                    osem,
                )
                for j in range(BCH)
            ]
            for cp in cps:
                cp.start()
            for cp in cps:
                cp.wait()

        idx_in(0, idx0)
        pltpu.async_copy(table_hbm.at[idx0], rows0, g0)

        @pl.loop(0, npair - 1)
        def _body(p):
            c = 2 * p
            idx_in(c + 1, idx1)
            pltpu.make_async_copy(table_hbm.at[idx0], rows0, g0).wait()
            pltpu.async_copy(table_hbm.at[idx1], rows1, g1)
            out_wr(c, rows0)
            idx_in(c + 2, idx0)
            pltpu.make_async_copy(table_hbm.at[idx1], rows1, g1).wait()
            pltpu.async_copy(table_hbm.at[idx0], rows0, g0)
            out_wr(c + 1, rows1)

        c = nchunk - 2
        idx_in(c + 1, idx1)
        pltpu.make_async_copy(table_hbm.at[idx0], rows0, g0).wait()
        pltpu.async_copy(table_hbm.at[idx1], rows1, g1)
        out_wr(c, rows0)
        pltpu.make_async_copy(table_hbm.at[idx1], rows1, g1).wait()
        out_wr(c + 1, rows1)

    return k(table, idx_flat)


@functools.partial(jax.jit, static_argnums=(2, 3, 4))
def _embed(indices, table, b, l, d):
    idx_flat = _tc_pad_idx(indices, b, l, table.shape[0])
    out = _sc_gather(idx_flat, table, b, l, d)
    return out[:, :l, :d]


def kernel(indices, table):
    b, l = indices.shape
    d = table.shape[1]
    return _embed(indices, table, b, l, d)
